# Initial kernel scaffold; baseline (speedup 1.0000x reference)
import jax
import jax.numpy as jnp
from jax.experimental import pallas as pl


def kernel(subgraph_x, subgraph_edge_index, subgraph_batch, W1, b1, W2, b2, Wc, bc):
    raise NotImplementedError("write your pallas kernel here")


# speedup vs baseline: 20.6367x; 20.6367x over previous
import functools

import jax
import jax.numpy as jnp
from jax import lax
from jax.experimental import pallas as pl
from jax.experimental.pallas import tpu as pltpu
from jax.experimental.pallas import tpu_sc as plsc

N = 10000
E = 320000
G = 512
D_IN = 128
D_HID = 128
D_OUT = 64
N_EXP = 16

NC = 2
NS = 16
NPAD = 10240


def _zero_vmem_1d(ref, n):
    def body(i, _):
        ref[pl.ds(i * 16, 16)] = jnp.zeros((16,), jnp.float32)
        return 0
    lax.fori_loop(0, n // 16, body, 0)


def _zero_vmem_2d(ref, rows, cols):
    per_row = cols // 16
    def body(i, _):
        r = i // per_row
        c = (i % per_row) * 16
        ref[r, pl.ds(c, 16)] = jnp.zeros((16,), jnp.float32)
        return 0
    lax.fori_loop(0, rows * per_row, body, 0)


def _make_hist_kernel():
    ept = E // (NC * NS)
    k = 1000
    rt = NPAD // NS
    mesh = plsc.VectorSubcoreMesh(core_axis_name="c", subcore_axis_name="s")

    @functools.partial(
        pl.kernel, mesh=mesh,
        out_type=jax.ShapeDtypeStruct((NC, NS, rt), jnp.float32),
        scratch_types=[
            pltpu.VMEM((k,), jnp.int32),
            pltpu.VMEM((k,), jnp.float32),
            pltpu.VMEM((rt,), jnp.float32),
            pltpu.VMEM_SHARED((NPAD,), jnp.float32),
        ],
    )
    def hist_kernel(dst_hbm, out_hbm, idx_v, ones_v, zer_v, hist_sh):
        c = lax.axis_index("c")
        s = lax.axis_index("s")
        _zero_vmem_1d(zer_v, rt)
        def ones_body(i, _):
            ones_v[pl.ds(i * 16, 16)] = jnp.ones((16,), jnp.float32)
            return 0
        lax.fori_loop(0, k // 16, ones_body, 0)
        pltpu.sync_copy(zer_v, hist_sh.at[pl.ds(s * rt, rt)])
        plsc.subcore_barrier()
        base = (c * NS + s) * ept
        def body(i, _):
            pltpu.sync_copy(dst_hbm.at[pl.ds(base + i * k, k)], idx_v)
            pltpu.sync_copy(ones_v, hist_sh.at[idx_v], add=True)
            return 0
        lax.fori_loop(0, ept // k, body, 0)
        plsc.subcore_barrier()
        pltpu.sync_copy(hist_sh.at[pl.ds(s * rt, rt)], out_hbm.at[c, s])

    return hist_kernel


def _make_scatter_kernel(d, k):
    ept = E // (NC * NS)
    rt = NPAD // NS
    zr = 128
    mesh = plsc.VectorSubcoreMesh(core_axis_name="c", subcore_axis_name="s")

    @functools.partial(
        pl.kernel, mesh=mesh,
        out_type=jax.ShapeDtypeStruct((NC, NPAD, d), jnp.float32),
        scratch_types=[
            pltpu.VMEM((k,), jnp.int32),
            pltpu.VMEM((k,), jnp.int32),
            pltpu.VMEM((k, d), jnp.float32),
            pltpu.VMEM((zr, d), jnp.float32),
            pltpu.VMEM_SHARED((NPAD, d), jnp.float32),
            pltpu.SemaphoreType.DMA,
        ],
    )
    def scatter_kernel(y_hbm, src_hbm, dst_hbm, out_hbm,
                       si_v, di_v, rows_v, z_v, acc_sh, sem):
        c = lax.axis_index("c")
        s = lax.axis_index("s")
        _zero_vmem_2d(z_v, zr, d)
        for j in range(rt // zr):
            pltpu.sync_copy(z_v, acc_sh.at[pl.ds(s * rt + j * zr, zr)])
        plsc.subcore_barrier()
        base = (c * NS + s) * ept
        def body(i, _):
            pltpu.sync_copy(src_hbm.at[pl.ds(base + i * k, k)], si_v)
            pltpu.sync_copy(dst_hbm.at[pl.ds(base + i * k, k)], di_v)
            pltpu.async_copy(y_hbm.at[si_v], rows_v, sem).wait()
            pltpu.sync_copy(rows_v, acc_sh.at[di_v], add=True)
            return 0
        lax.fori_loop(0, ept // k, body, 0)
        plsc.subcore_barrier()
        pltpu.sync_copy(acc_sh.at[pl.ds(s * rt, rt)],
                        out_hbm.at[c, pl.ds(s * rt, rt)])

    return scatter_kernel


_hist_call = _make_hist_kernel()
_scatter128 = _make_scatter_kernel(128, 200)


def _tc_stage1(x, w1, b1row, histcol):
    def body(x_ref, w_ref, b_ref, h_ref, y_ref, t_ref, dinv_ref):
        xw = jnp.dot(x_ref[...], w_ref[...], preferred_element_type=jnp.float32)
        dinv = lax.rsqrt(h_ref[...] + 1.0)
        y_ref[...] = xw * dinv
        t_ref[...] = xw * (dinv * dinv) + b_ref[...]
        dinv_ref[...] = dinv

    return pl.pallas_call(
        body,
        out_shape=[
            jax.ShapeDtypeStruct((N, D_HID), jnp.float32),
            jax.ShapeDtypeStruct((N, D_HID), jnp.float32),
            jax.ShapeDtypeStruct((N, 1), jnp.float32),
        ],
    )(x, w1, b1row, histcol)


def _tc_stage2(s1p, t1, dinv, w2, b2row):
    def body(sp_ref, t1_ref, dinv_ref, w_ref, b_ref, yt_ref):
        h = dinv_ref[...] * (sp_ref[0, :N] + sp_ref[1, :N]) + t1_ref[...]
        xw = jnp.dot(h, w_ref[...], preferred_element_type=jnp.float32)
        dinv = dinv_ref[...]
        yt_ref[...] = jnp.concatenate(
            [xw * dinv, xw * (dinv * dinv) + b_ref[...]], axis=1)

    return pl.pallas_call(
        body,
        out_shape=jax.ShapeDtypeStruct((N, 2 * D_OUT), jnp.float32),
    )(s1p, t1, dinv, w2, b2row)


def _tc_stage3(s2p, yt2, dinv, batchrow, wc, bcrow):
    blk = 1000
    grid = N // blk

    def body(sp_ref, yt_ref, dinv_ref, b_ref, wc_ref, bc_ref, out_ref,
             pool_acc, cnt_acc):
        i = pl.program_id(0)

        @pl.when(i == 0)
        def _():
            pool_acc[...] = jnp.zeros_like(pool_acc)
            cnt_acc[...] = jnp.zeros_like(cnt_acc)

        out2 = (dinv_ref[...] * (sp_ref[0, :, :D_OUT] + sp_ref[1, :, :D_OUT])
                + yt_ref[:, D_OUT:])
        gids = lax.broadcasted_iota(jnp.int32, (G, blk), 0)
        mask = (b_ref[0] == gids).astype(jnp.float32)
        pool_acc[...] += jnp.dot(mask, out2,
                                 preferred_element_type=jnp.float32)
        cnt_acc[...] += jnp.sum(mask, axis=1, keepdims=True)

        @pl.when(i == pl.num_programs(0) - 1)
        def _():
            pooled = pool_acc[...] / jnp.maximum(cnt_acc[...], 1.0)
            logits = jnp.dot(pooled, wc_ref[...],
                             preferred_element_type=jnp.float32) + bc_ref[...]
            m = jnp.max(logits, axis=1, keepdims=True)
            e = jnp.exp(logits - m)
            out_ref[...] = e / jnp.sum(e, axis=1, keepdims=True)

    return pl.pallas_call(
        body,
        grid=(grid,),
        in_specs=[
            pl.BlockSpec((2, blk, 2 * D_OUT), lambda i: (0, i, 0)),
            pl.BlockSpec((blk, 2 * D_OUT), lambda i: (i, 0)),
            pl.BlockSpec((blk, 1), lambda i: (i, 0)),
            pl.BlockSpec((1, 1, blk), lambda i: (i, 0, 0)),
            pl.BlockSpec((D_OUT, N_EXP), lambda i: (0, 0)),
            pl.BlockSpec((1, N_EXP), lambda i: (0, 0)),
        ],
        out_specs=pl.BlockSpec((G, N_EXP), lambda i: (0, 0)),
        out_shape=jax.ShapeDtypeStruct((G, N_EXP), jnp.float32),
        scratch_shapes=[
            pltpu.VMEM((G, D_OUT), jnp.float32),
            pltpu.VMEM((G, 1), jnp.float32),
        ],
    )(s2p, yt2, dinv, batchrow, wc, bcrow)


def kernel(subgraph_x, subgraph_edge_index, subgraph_batch,
           W1, b1, W2, b2, Wc, bc):
    src = subgraph_edge_index[0]
    dst = subgraph_edge_index[1]

    hist_parts = _hist_call(dst)
    histcol = (hist_parts.reshape(NC, NPAD).sum(axis=0)[:N]
               .reshape(N, 1))

    y1, t1, dinv = _tc_stage1(subgraph_x, W1, b1.reshape(1, D_HID), histcol)
    s1p = _scatter128(y1, src, dst)
    yt2 = _tc_stage2(s1p, t1, dinv, W2, b2.reshape(1, D_OUT))
    s2p = _scatter128(yt2, src, dst)
    return _tc_stage3(s2p, yt2, dinv, subgraph_batch.reshape(N // 1000, 1, 1000),
                      Wc, bc.reshape(1, N_EXP))


# speedup vs baseline: 31.9984x; 1.5506x over previous
import functools

import jax
import jax.numpy as jnp
from jax import lax
from jax.experimental import pallas as pl
from jax.experimental.pallas import tpu as pltpu
from jax.experimental.pallas import tpu_sc as plsc

N = 10000
E = 320000
G = 512
D_IN = 128
D_HID = 128
D_OUT = 64
N_EXP = 16

NC = 2
NS = 16
NPAD = 10240


def _zero_vmem_1d(ref, n):
    def body(i, _):
        ref[pl.ds(i * 16, 16)] = jnp.zeros((16,), jnp.float32)
        return 0
    lax.fori_loop(0, n // 16, body, 0)


def _zero_vmem_2d(ref, rows, cols):
    per_row = cols // 16
    def body(i, _):
        r = i // per_row
        c = (i % per_row) * 16
        ref[r, pl.ds(c, 16)] = jnp.zeros((16,), jnp.float32)
        return 0
    lax.fori_loop(0, rows * per_row, body, 0)


def _make_hist_kernel():
    ept = E // (NC * NS)
    k = 1000
    rt = NPAD // NS
    mesh = plsc.VectorSubcoreMesh(core_axis_name="c", subcore_axis_name="s")

    @functools.partial(
        pl.kernel, mesh=mesh,
        out_type=jax.ShapeDtypeStruct((NC, NS, rt), jnp.float32),
        scratch_types=[
            pltpu.VMEM((k,), jnp.int32),
            pltpu.VMEM((k,), jnp.float32),
            pltpu.VMEM((rt,), jnp.float32),
            pltpu.VMEM_SHARED((NPAD,), jnp.float32),
        ],
    )
    def hist_kernel(dst_hbm, out_hbm, idx_v, ones_v, zer_v, hist_sh):
        c = lax.axis_index("c")
        s = lax.axis_index("s")
        _zero_vmem_1d(zer_v, rt)
        def ones_body(i, _):
            ones_v[pl.ds(i * 16, 16)] = jnp.ones((16,), jnp.float32)
            return 0
        lax.fori_loop(0, k // 16, ones_body, 0)
        pltpu.sync_copy(zer_v, hist_sh.at[pl.ds(s * rt, rt)])
        plsc.subcore_barrier()
        base = (c * NS + s) * ept
        def body(i, _):
            pltpu.sync_copy(dst_hbm.at[pl.ds(base + i * k, k)], idx_v)
            pltpu.sync_copy(ones_v, hist_sh.at[idx_v], add=True)
            return 0
        lax.fori_loop(0, ept // k, body, 0)
        plsc.subcore_barrier()
        pltpu.sync_copy(hist_sh.at[pl.ds(s * rt, rt)], out_hbm.at[c, s])

    return hist_kernel


def _make_scatter_kernel(d, k):
    ept = E // (NC * NS)
    nch = ept // k
    tail = ept - nch * k
    rt = NPAD // NS
    zr = 128
    mesh = plsc.VectorSubcoreMesh(core_axis_name="c", subcore_axis_name="s")

    @functools.partial(
        pl.kernel, mesh=mesh,
        out_type=jax.ShapeDtypeStruct((NC, NPAD, d), jnp.float32),
        scratch_types=[
            pltpu.VMEM((k,), jnp.int32),
            pltpu.VMEM((k,), jnp.int32),
            pltpu.VMEM((k,), jnp.int32),
            pltpu.VMEM((k,), jnp.int32),
            pltpu.VMEM((k, d), jnp.float32),
            pltpu.VMEM((k, d), jnp.float32),
            pltpu.VMEM((64,), jnp.int32),
            pltpu.VMEM((64,), jnp.int32),
            pltpu.VMEM_SHARED((NPAD, d), jnp.float32),
            pltpu.SemaphoreType.DMA,
            pltpu.SemaphoreType.DMA,
            pltpu.SemaphoreType.DMA,
            pltpu.SemaphoreType.DMA,
        ],
    )
    def scatter_kernel(y_hbm, src_hbm, dst_hbm, out_hbm,
                       si0_v, si1_v, di0_v, di1_v, rows0_v, rows1_v,
                       st_v, dt_v,
                       acc_sh, semg0, semg1, semi0, semi1):
        c = lax.axis_index("c")
        s = lax.axis_index("s")
        si = (si0_v, si1_v)
        di = (di0_v, di1_v)
        rows = (rows0_v, rows1_v)
        semg = (semg0, semg1)
        semi = (semi0, semi1)
        base = (c * NS + s) * ept

        def issue_idx(b, j):
            pltpu.async_copy(src_hbm.at[pl.ds(base + j * k, k)], si[b], semi[b])
            pltpu.async_copy(dst_hbm.at[pl.ds(base + j * k, k)], di[b], semi[b])

        def wait_idx(b):
            pltpu.make_async_copy(src_hbm.at[pl.ds(0, k)], si[b],
                                  semi[b]).wait()
            pltpu.make_async_copy(src_hbm.at[pl.ds(0, k)], di[b],
                                  semi[b]).wait()

        issue_idx(0, 0)
        issue_idx(1, 1)
        _zero_vmem_2d(rows0_v, zr, d)
        for j in range(rt // zr):
            pltpu.sync_copy(rows0_v.at[pl.ds(0, zr)],
                            acc_sh.at[pl.ds(s * rt + j * zr, zr)])
        plsc.subcore_barrier()
        if tail:
            pltpu.sync_copy(src_hbm.at[pl.ds(base + nch * k, tail)], st_v)
            pltpu.sync_copy(dst_hbm.at[pl.ds(base + nch * k, tail)], dt_v)
            pltpu.async_copy(y_hbm.at[st_v], rows0_v.at[pl.ds(0, tail)],
                             semg0).wait()
            pltpu.sync_copy(rows0_v.at[pl.ds(0, tail)], acc_sh.at[dt_v],
                            add=True)
        wait_idx(0)
        pltpu.async_copy(y_hbm.at[si0_v], rows0_v, semg0)

        def body(i, _):
            j0 = i * 2
            for b in range(2):
                j = j0 + b
                o = 1 - b

                @pl.when(j + 1 < nch)
                def _():
                    wait_idx(o)
                    pltpu.async_copy(y_hbm.at[si[o]], rows[o], semg[o])
                pltpu.make_async_copy(y_hbm.at[pl.ds(0, k)],
                                      rows[b], semg[b]).wait()
                pltpu.sync_copy(rows[b], acc_sh.at[di[b]], add=True)

                @pl.when(j + 2 < nch)
                def _():
                    issue_idx(b, j + 2)
            return 0
        lax.fori_loop(0, nch // 2, body, 0)
        plsc.subcore_barrier()
        pltpu.sync_copy(acc_sh.at[pl.ds(s * rt, rt)],
                        out_hbm.at[c, pl.ds(s * rt, rt)])

    return scatter_kernel


KCH = 184
_hist_call = _make_hist_kernel()
_scatter128 = _make_scatter_kernel(128, KCH)


def _tc_stage1(x, w1, b1row, histcol):
    def body(x_ref, w_ref, b_ref, h_ref, y_ref, t_ref, dinv_ref):
        xw = jnp.dot(x_ref[...], w_ref[...], preferred_element_type=jnp.float32)
        dinv = lax.rsqrt(h_ref[...] + 1.0)
        y_ref[...] = xw * dinv
        t_ref[...] = xw * (dinv * dinv) + b_ref[...]
        dinv_ref[...] = dinv

    return pl.pallas_call(
        body,
        out_shape=[
            jax.ShapeDtypeStruct((N, D_HID), jnp.float32),
            jax.ShapeDtypeStruct((N, D_HID), jnp.float32),
            jax.ShapeDtypeStruct((N, 1), jnp.float32),
        ],
    )(x, w1, b1row, histcol)


def _tc_stage2(s1p, t1, dinv, w2, b2row):
    def body(sp_ref, t1_ref, dinv_ref, w_ref, b_ref, yt_ref):
        h = dinv_ref[...] * (sp_ref[0, :N] + sp_ref[1, :N]) + t1_ref[...]
        xw = jnp.dot(h, w_ref[...], preferred_element_type=jnp.float32)
        dinv = dinv_ref[...]
        yt_ref[...] = jnp.concatenate(
            [xw * dinv, xw * (dinv * dinv) + b_ref[...]], axis=1)

    return pl.pallas_call(
        body,
        out_shape=jax.ShapeDtypeStruct((N, 2 * D_OUT), jnp.float32),
    )(s1p, t1, dinv, w2, b2row)


def _tc_stage3(s2p, yt2, dinv, batchrow, wc, bcrow):
    blk = 1000
    grid = N // blk

    def body(sp_ref, yt_ref, dinv_ref, b_ref, wc_ref, bc_ref, out_ref,
             pool_acc, cnt_acc):
        i = pl.program_id(0)

        @pl.when(i == 0)
        def _():
            pool_acc[...] = jnp.zeros_like(pool_acc)
            cnt_acc[...] = jnp.zeros_like(cnt_acc)

        out2 = (dinv_ref[...] * (sp_ref[0, :, :D_OUT] + sp_ref[1, :, :D_OUT])
                + yt_ref[:, D_OUT:])
        gids = lax.broadcasted_iota(jnp.int32, (G, blk), 0)
        mask = (b_ref[0] == gids).astype(jnp.float32)
        pool_acc[...] += jnp.dot(mask, out2,
                                 preferred_element_type=jnp.float32)
        cnt_acc[...] += jnp.sum(mask, axis=1, keepdims=True)

        @pl.when(i == pl.num_programs(0) - 1)
        def _():
            pooled = pool_acc[...] / jnp.maximum(cnt_acc[...], 1.0)
            logits = jnp.dot(pooled, wc_ref[...],
                             preferred_element_type=jnp.float32) + bc_ref[...]
            m = jnp.max(logits, axis=1, keepdims=True)
            e = jnp.exp(logits - m)
            out_ref[...] = e / jnp.sum(e, axis=1, keepdims=True)

    return pl.pallas_call(
        body,
        grid=(grid,),
        in_specs=[
            pl.BlockSpec((2, blk, 2 * D_OUT), lambda i: (0, i, 0)),
            pl.BlockSpec((blk, 2 * D_OUT), lambda i: (i, 0)),
            pl.BlockSpec((blk, 1), lambda i: (i, 0)),
            pl.BlockSpec((1, 1, blk), lambda i: (i, 0, 0)),
            pl.BlockSpec((D_OUT, N_EXP), lambda i: (0, 0)),
            pl.BlockSpec((1, N_EXP), lambda i: (0, 0)),
        ],
        out_specs=pl.BlockSpec((G, N_EXP), lambda i: (0, 0)),
        out_shape=jax.ShapeDtypeStruct((G, N_EXP), jnp.float32),
        scratch_shapes=[
            pltpu.VMEM((G, D_OUT), jnp.float32),
            pltpu.VMEM((G, 1), jnp.float32),
        ],
    )(s2p, yt2, dinv, batchrow, wc, bcrow)


def kernel(subgraph_x, subgraph_edge_index, subgraph_batch,
           W1, b1, W2, b2, Wc, bc):
    src = subgraph_edge_index[0]
    dst = subgraph_edge_index[1]

    hist_parts = _hist_call(dst)
    histcol = (hist_parts.reshape(NC, NPAD).sum(axis=0)[:N]
               .reshape(N, 1))

    y1, t1, dinv = _tc_stage1(subgraph_x, W1, b1.reshape(1, D_HID), histcol)
    s1p = _scatter128(y1, src, dst)
    yt2 = _tc_stage2(s1p, t1, dinv, W2, b2.reshape(1, D_OUT))
    s2p = _scatter128(yt2, src, dst)
    return _tc_stage3(s2p, yt2, dinv, subgraph_batch.reshape(N // 1000, 1, 1000),
                      Wc, bc.reshape(1, N_EXP))


# speedup vs baseline: 32.0562x; 1.0018x over previous
import functools

import jax
import jax.numpy as jnp
from jax import lax
from jax.experimental import pallas as pl
from jax.experimental.pallas import tpu as pltpu
from jax.experimental.pallas import tpu_sc as plsc

N = 10000
E = 320000
G = 512
D_IN = 128
D_HID = 128
D_OUT = 64
N_EXP = 16

NC = 2
NS = 16
NPAD = 10240


def _zero_vmem_1d(ref, n):
    def body(i, _):
        ref[pl.ds(i * 16, 16)] = jnp.zeros((16,), jnp.float32)
        return 0
    lax.fori_loop(0, n // 16, body, 0)


def _zero_vmem_2d(ref, rows, cols):
    per_row = cols // 16
    def body(i, _):
        r = i // per_row
        c = (i % per_row) * 16
        ref[r, pl.ds(c, 16)] = jnp.zeros((16,), jnp.float32)
        return 0
    lax.fori_loop(0, rows * per_row, body, 0)


def _make_hist_kernel():
    ept = E // (NC * NS)
    k = 1000
    rt = NPAD // NS
    mesh = plsc.VectorSubcoreMesh(core_axis_name="c", subcore_axis_name="s")

    @functools.partial(
        pl.kernel, mesh=mesh,
        out_type=jax.ShapeDtypeStruct((NC, NS, rt), jnp.float32),
        scratch_types=[
            pltpu.VMEM((k,), jnp.int32),
            pltpu.VMEM((k,), jnp.float32),
            pltpu.VMEM((rt,), jnp.float32),
            pltpu.VMEM_SHARED((NPAD,), jnp.float32),
        ],
    )
    def hist_kernel(dst_hbm, out_hbm, idx_v, ones_v, zer_v, hist_sh):
        c = lax.axis_index("c")
        s = lax.axis_index("s")
        _zero_vmem_1d(zer_v, rt)
        def ones_body(i, _):
            ones_v[pl.ds(i * 16, 16)] = jnp.ones((16,), jnp.float32)
            return 0
        lax.fori_loop(0, k // 16, ones_body, 0)
        pltpu.sync_copy(zer_v, hist_sh.at[pl.ds(s * rt, rt)])
        plsc.subcore_barrier()
        base = (c * NS + s) * ept
        def body(i, _):
            pltpu.sync_copy(dst_hbm.at[pl.ds(base + i * k, k)], idx_v)
            pltpu.sync_copy(ones_v, hist_sh.at[idx_v], add=True)
            return 0
        lax.fori_loop(0, ept // k, body, 0)
        plsc.subcore_barrier()
        pltpu.sync_copy(hist_sh.at[pl.ds(s * rt, rt)], out_hbm.at[c, s])

    return hist_kernel


def _make_scatter_kernel(d, k):
    ept = E // (NC * NS)
    nch = ept // k
    tail = ept - nch * k
    rt = NPAD // NS
    zr = 128
    mesh = plsc.VectorSubcoreMesh(core_axis_name="c", subcore_axis_name="s")

    @functools.partial(
        pl.kernel, mesh=mesh,
        out_type=jax.ShapeDtypeStruct((NC, NPAD, d), jnp.float32),
        scratch_types=[
            pltpu.VMEM((k,), jnp.int32),
            pltpu.VMEM((k,), jnp.int32),
            pltpu.VMEM((k,), jnp.int32),
            pltpu.VMEM((k,), jnp.int32),
            pltpu.VMEM((k, d), jnp.float32),
            pltpu.VMEM((k, d), jnp.float32),
            pltpu.VMEM((64,), jnp.int32),
            pltpu.VMEM((64,), jnp.int32),
            pltpu.VMEM_SHARED((NPAD, d), jnp.float32),
            pltpu.SemaphoreType.DMA,
            pltpu.SemaphoreType.DMA,
            pltpu.SemaphoreType.DMA,
            pltpu.SemaphoreType.DMA,
        ],
    )
    def scatter_kernel(y_hbm, src_hbm, dst_hbm, out_hbm,
                       si0_v, si1_v, di0_v, di1_v, rows0_v, rows1_v,
                       st_v, dt_v,
                       acc_sh, semg0, semg1, semi0, semi1):
        c = lax.axis_index("c")
        s = lax.axis_index("s")
        si = (si0_v, si1_v)
        di = (di0_v, di1_v)
        rows = (rows0_v, rows1_v)
        semg = (semg0, semg1)
        semi = (semi0, semi1)
        base = (c * NS + s) * ept

        def issue_idx(b, j):
            pltpu.async_copy(src_hbm.at[pl.ds(base + j * k, k)], si[b], semi[b])
            pltpu.async_copy(dst_hbm.at[pl.ds(base + j * k, k)], di[b], semi[b])

        def wait_idx(b):
            pltpu.make_async_copy(src_hbm.at[pl.ds(0, k)], si[b],
                                  semi[b]).wait()
            pltpu.make_async_copy(src_hbm.at[pl.ds(0, k)], di[b],
                                  semi[b]).wait()

        issue_idx(0, 0)
        issue_idx(1, 1)
        _zero_vmem_2d(rows0_v, zr, d)
        for j in range(rt // zr):
            pltpu.sync_copy(rows0_v.at[pl.ds(0, zr)],
                            acc_sh.at[pl.ds(s * rt + j * zr, zr)])
        plsc.subcore_barrier()
        if tail:
            pltpu.sync_copy(src_hbm.at[pl.ds(base + nch * k, tail)], st_v)
            pltpu.sync_copy(dst_hbm.at[pl.ds(base + nch * k, tail)], dt_v)
            pltpu.async_copy(y_hbm.at[st_v], rows0_v.at[pl.ds(0, tail)],
                             semg0).wait()
            pltpu.sync_copy(rows0_v.at[pl.ds(0, tail)], acc_sh.at[dt_v],
                            add=True)
        wait_idx(0)
        pltpu.async_copy(y_hbm.at[si0_v], rows0_v, semg0)

        def body(i, _):
            j0 = i * 2
            for b in range(2):
                j = j0 + b
                o = 1 - b

                @pl.when(j + 1 < nch)
                def _():
                    wait_idx(o)
                    pltpu.async_copy(y_hbm.at[si[o]], rows[o], semg[o])
                pltpu.make_async_copy(y_hbm.at[pl.ds(0, k)],
                                      rows[b], semg[b]).wait()
                pltpu.sync_copy(rows[b], acc_sh.at[di[b]], add=True)

                @pl.when(j + 2 < nch)
                def _():
                    issue_idx(b, j + 2)
            return 0
        lax.fori_loop(0, nch // 2, body, 0)
        plsc.subcore_barrier()
        pltpu.sync_copy(acc_sh.at[pl.ds(s * rt, rt)],
                        out_hbm.at[c, pl.ds(s * rt, rt)])

    return scatter_kernel


KCH = 184
_hist_call = _make_hist_kernel()
_scatter128 = _make_scatter_kernel(128, KCH)


def _tc_stage0(x, w1):
    def body(x_ref, w_ref, xw_ref):
        xw_ref[...] = jnp.dot(x_ref[...], w_ref[...],
                              preferred_element_type=jnp.float32)

    return pl.pallas_call(
        body,
        out_shape=jax.ShapeDtypeStruct((N, D_HID), jnp.float32),
    )(x, w1)


def _tc_stage1(xw1, b1row, histcol):
    def body(xw_ref, b_ref, h_ref, y_ref, t_ref, dinv_ref):
        xw = xw_ref[...]
        dinv = lax.rsqrt(h_ref[...] + 1.0)
        y_ref[...] = xw * dinv
        t_ref[...] = xw * (dinv * dinv) + b_ref[...]
        dinv_ref[...] = dinv

    return pl.pallas_call(
        body,
        out_shape=[
            jax.ShapeDtypeStruct((N, D_HID), jnp.float32),
            jax.ShapeDtypeStruct((N, D_HID), jnp.float32),
            jax.ShapeDtypeStruct((N, 1), jnp.float32),
        ],
    )(xw1, b1row, histcol)


def _tc_stage2(s1p, t1, dinv, w2, b2row):
    def body(sp_ref, t1_ref, dinv_ref, w_ref, b_ref, yt_ref):
        h = dinv_ref[...] * (sp_ref[0, :N] + sp_ref[1, :N]) + t1_ref[...]
        xw = jnp.dot(h, w_ref[...], preferred_element_type=jnp.float32)
        dinv = dinv_ref[...]
        yt_ref[...] = jnp.concatenate(
            [xw * dinv, xw * (dinv * dinv) + b_ref[...]], axis=1)

    return pl.pallas_call(
        body,
        out_shape=jax.ShapeDtypeStruct((N, 2 * D_OUT), jnp.float32),
    )(s1p, t1, dinv, w2, b2row)


def _tc_stage3(s2p, yt2, dinv, batchrow, wc, bcrow):
    blk = 2000
    grid = N // blk

    def body(sp_ref, yt_ref, dinv_ref, b_ref, wc_ref, bc_ref, out_ref,
             pool_acc, cnt_acc):
        i = pl.program_id(0)

        @pl.when(i == 0)
        def _():
            pool_acc[...] = jnp.zeros_like(pool_acc)
            cnt_acc[...] = jnp.zeros_like(cnt_acc)

        out2 = (dinv_ref[...] * (sp_ref[0, :, :D_OUT] + sp_ref[1, :, :D_OUT])
                + yt_ref[:, D_OUT:])
        gids = lax.broadcasted_iota(jnp.int32, (G, blk), 0)
        mask = (b_ref[0] == gids).astype(jnp.float32)
        pool_acc[...] += jnp.dot(mask, out2,
                                 preferred_element_type=jnp.float32)
        cnt_acc[...] += jnp.sum(mask, axis=1, keepdims=True)

        @pl.when(i == pl.num_programs(0) - 1)
        def _():
            pooled = pool_acc[...] / jnp.maximum(cnt_acc[...], 1.0)
            logits = jnp.dot(pooled, wc_ref[...],
                             preferred_element_type=jnp.float32) + bc_ref[...]
            m = jnp.max(logits, axis=1, keepdims=True)
            e = jnp.exp(logits - m)
            out_ref[...] = e / jnp.sum(e, axis=1, keepdims=True)

    return pl.pallas_call(
        body,
        grid=(grid,),
        in_specs=[
            pl.BlockSpec((2, blk, 2 * D_OUT), lambda i: (0, i, 0)),
            pl.BlockSpec((blk, 2 * D_OUT), lambda i: (i, 0)),
            pl.BlockSpec((blk, 1), lambda i: (i, 0)),
            pl.BlockSpec((1, 1, blk), lambda i: (i, 0, 0)),
            pl.BlockSpec((D_OUT, N_EXP), lambda i: (0, 0)),
            pl.BlockSpec((1, N_EXP), lambda i: (0, 0)),
        ],
        out_specs=pl.BlockSpec((G, N_EXP), lambda i: (0, 0)),
        out_shape=jax.ShapeDtypeStruct((G, N_EXP), jnp.float32),
        scratch_shapes=[
            pltpu.VMEM((G, D_OUT), jnp.float32),
            pltpu.VMEM((G, 1), jnp.float32),
        ],
    )(s2p, yt2, dinv, batchrow, wc, bcrow)


def kernel(subgraph_x, subgraph_edge_index, subgraph_batch,
           W1, b1, W2, b2, Wc, bc):
    src = subgraph_edge_index[0]
    dst = subgraph_edge_index[1]

    xw1 = _tc_stage0(subgraph_x, W1)
    hist_parts = _hist_call(dst)
    histcol = (hist_parts.reshape(NC, NPAD).sum(axis=0)[:N]
               .reshape(N, 1))

    y1, t1, dinv = _tc_stage1(xw1, b1.reshape(1, D_HID), histcol)
    s1p = _scatter128(y1, src, dst)
    yt2 = _tc_stage2(s1p, t1, dinv, W2, b2.reshape(1, D_OUT))
    s2p = _scatter128(yt2, src, dst)
    return _tc_stage3(s2p, yt2, dinv, subgraph_batch.reshape(N // 2000, 1, 2000),
                      Wc, bc.reshape(1, N_EXP))
